# baseline (speedup 1.0000x reference)
import functools

import jax
import jax.numpy as jnp
import numpy as np
from jax import lax
from jax.experimental import pallas as pl
from jax.experimental.pallas import tpu as pltpu

_IGNORE_LABEL = 255
_MAX_N_RATIO = 1.0 / 100
_LB_SMOOTH = 0.2
_MAX_CLIP_DIST = 20.0

_B = 4
_H = 224
_W = 224
_C = 19
_NDIR = 7

_NEPS = 160


def _build_eps_table():
    out = []
    e = np.float32(1e-5)
    for _ in range(_NEPS):
        out.append(float(e))
        e = np.float32(e * np.float32(1.2))
    return out


_EPS_TABLE = _build_eps_table()

_BIG = 1.0e9

_DX = (1, -1, 0, 0, -1, 1, -1, 1, 0)
_DY = (0, 0, -1, 1, 1, 1, -1, -1, 0)

_KUNROLL = 8


def _pad2d(x, value):
    h, w = x.shape
    row = jnp.full((1, w), value, dtype=x.dtype)
    x = jnp.concatenate([row, x, row], axis=0)
    col = jnp.full((h + 2, 1), value, dtype=x.dtype)
    return jnp.concatenate([col, x, col], axis=1)


def _pad3d(x, value):
    b, h, w = x.shape
    row = jnp.full((b, 1, w), value, dtype=x.dtype)
    x = jnp.concatenate([row, x, row], axis=1)
    col = jnp.full((b, h + 2, 1), value, dtype=x.dtype)
    return jnp.concatenate([col, x, col], axis=2)


def _dist_kernel(tg_ref, dist_ref, dgt_ref, g2t_ref):
    gt = tg_ref[...].astype(jnp.int32)

    dud = gt[:, 1:, :] - gt[:, :-1, :]
    dlr = gt[:, :, 1:] - gt[:, :, :-1]
    zrow = jnp.zeros((_B, 1, _W), dtype=jnp.int32)
    zcol = jnp.zeros((_B, _H, 1), dtype=jnp.int32)
    b_ud = jnp.concatenate([dud, zrow], axis=1) != 0
    b_lr = jnp.concatenate([dlr, zcol], axis=2) != 0
    bnd = b_ud | b_lr | (gt == _IGNORE_LABEL)
    pos = jnp.logical_not(bnd)

    init = jnp.where(pos, 1.0e6, 0.0).astype(jnp.float32)
    fwd = init
    bwd = init
    t = 1
    while t < _H:
        pad = jnp.full((_B, t, _W), _BIG, dtype=jnp.float32)
        fwd = jnp.minimum(
            fwd, jnp.concatenate([pad, fwd[:, :-t, :]], axis=1) + t)
        bwd = jnp.minimum(
            bwd, jnp.concatenate([bwd[:, t:, :], pad], axis=1) + t)
        t *= 2
    d0 = jnp.minimum(fwd, bwd)
    g2 = d0 * d0

    g2t_ref[...] = jnp.concatenate(
        [jnp.transpose(g2[b]) for b in range(_B)], axis=1)
    jf = lax.broadcasted_iota(jnp.int32, (_W, 1), 0).astype(jnp.float32)

    def body(tk, acc):
        k0 = pl.multiple_of(tk * _KUNROLL, _KUNROLL)
        rows = g2t_ref[pl.ds(k0, _KUNROLL), :]
        k0f = k0.astype(jnp.float32)
        for u in range(_KUNROLL):
            off = (jf - (k0f + u)) ** 2
            acc = jnp.minimum(acc, off + rows[u:u + 1, :])
        return acc

    acc0 = jnp.full((_W, _B * _H), 1.0e30, dtype=jnp.float32)
    d2t = lax.fori_loop(0, _W // _KUNROLL, body, acc0)
    dist = jnp.stack(
        [jnp.transpose(jnp.sqrt(d2t[:, b * _H:(b + 1) * _H]))
         for b in range(_B)], axis=0)

    dmap = jnp.where(pos, jnp.maximum(dist - 1.0, 0.0), 0.0)

    pdm = _pad3d(dmap, 1.0e5)
    best = pdm[:, 1 + _DX[0]:1 + _DX[0] + _H, 1 + _DY[0]:1 + _DY[0] + _W]
    idx = jnp.zeros((_B, _H, _W), dtype=jnp.int32)
    for d in range(1, 9):
        v = pdm[:, 1 + _DX[d]:1 + _DX[d] + _H, 1 + _DY[d]:1 + _DY[d] + _W]
        better = v < best
        idx = jnp.where(better, jnp.int32(d), idx)
        best = jnp.minimum(best, v)

    dist_ref[...] = dmap
    dgt_ref[...] = idx


def _kl_kernel(lg_ref, dist_ref, dgt_ref, klc_ref, contrib_ref):
    x = lg_ref[0]

    m = x[0]
    for c in range(1, _C):
        m = jnp.maximum(m, x[c])
    sh = x - m[None]
    e = jnp.exp(sh)
    se = e[0]
    for c in range(1, _C):
        se = se + e[c]
    ls = sh - jnp.log(se)[None]
    pk = jnp.exp(ls)

    kud = jnp.zeros((_H - 1, _W), dtype=jnp.float32)
    klr = jnp.zeros((_H, _W - 1), dtype=jnp.float32)
    for c in range(_C):
        kud = kud + pk[c, :-1, :] * (ls[c, :-1, :] - ls[c, 1:, :])
        klr = klr + pk[c, :, :-1] * (ls[c, :, :-1] - ls[c, :, 1:])
    zrow = jnp.zeros((1, _W), dtype=jnp.float32)
    zcol = jnp.zeros((_H, 1), dtype=jnp.float32)
    klc_ref[0] = (jnp.concatenate([kud, zrow], axis=0)
                  + jnp.concatenate([klr, zcol], axis=1))

    unif = 1.0 / _C
    lunif = -float(np.log(np.float32(_C)))
    s = pk[0] * ls[0]
    for c in range(1, _C):
        s = s + pk[c] * ls[c]
    spad = _pad2d(s, lunif)
    h2 = _H + 2
    prow = jnp.full((_C, 1, _W), unif, dtype=jnp.float32)
    ppad = jnp.concatenate([prow, pk, prow], axis=1)
    pcol = jnp.full((_C, h2, 1), unif, dtype=jnp.float32)
    ppad = jnp.concatenate([pcol, ppad, pcol], axis=2)

    dps = []
    for d in range(_NDIR):
        dx, dy = _DX[d], _DY[d]
        sn = spad[1 + dx:1 + dx + _H, 1 + dy:1 + dy + _W]
        dot = jnp.zeros((_H, _W), dtype=jnp.float32)
        for c in range(_C):
            dot = dot + ppad[c, 1 + dx:1 + dx + _H, 1 + dy:1 + dy + _W] * ls[c]
        dps.append(sn - dot)

    m7 = dps[0]
    for d in range(1, _NDIR):
        m7 = jnp.maximum(m7, dps[d])
    se7 = jnp.zeros((_H, _W), dtype=jnp.float32)
    for d in range(_NDIR):
        se7 = se7 + jnp.exp(dps[d] - m7)
    lse7 = jnp.log(se7)
    dgt = dgt_ref[0]
    sumlogs = jnp.zeros((_H, _W), dtype=jnp.float32)
    pick = jnp.zeros((_H, _W), dtype=jnp.float32)
    for d in range(_NDIR):
        logs_d = (dps[d] - m7) - lse7
        sumlogs = sumlogs + logs_d
        pick = pick + jnp.where(dgt == d, logs_d, 0.0)
    lb_pos = 1.0 - _LB_SMOOTH
    lb_neg = _LB_SMOOTH / _NDIR
    loss_vec = -(lb_neg * sumlogs + (lb_pos - lb_neg) * pick)
    w = jnp.clip(dist_ref[0], 0.0, _MAX_CLIP_DIST) / _MAX_CLIP_DIST
    contrib_ref[0] = loss_vec * w


def _loss_kernel(tbl_ref, klc_ref, contrib_ref, dgt_ref, out_ref):
    klc = klc_ref[...]

    max_n = _H * _W * _MAX_N_RATIO

    def bs_body(_, lohi):
        lo, hi = lohi
        mid = (lo + hi) // 2
        cnt = jnp.sum((klc > tbl_ref[0, mid]).astype(jnp.float32))
        big = cnt > max_n
        return (jnp.where(big, mid + 1, lo), jnp.where(big, hi, mid))

    lo, _hi = lax.fori_loop(
        0, 8, bs_body, (jnp.int32(0), jnp.int32(_NEPS)))
    eps = tbl_ref[0, jnp.minimum(lo, _NEPS - 1)]

    bp = _pad3d((klc > eps).astype(jnp.float32), 0.0)
    edge = jnp.zeros((_B, _H, _W), dtype=jnp.float32)
    for dx in (-1, 0, 1):
        for dy in (-1, 0, 1):
            edge = edge + bp[:, 1 + dx:1 + dx + _H, 1 + dy:1 + dy + _W]
    pred = edge > 0.0

    sel = jnp.logical_and(pred, dgt_ref[...] != 8).astype(jnp.float32)
    num = jnp.sum(contrib_ref[...] * sel)
    den = jnp.sum(sel)
    npred = jnp.sum(pred.astype(jnp.float32))

    lane = lax.broadcasted_iota(jnp.int32, (1, 128), 1)
    out_ref[...] = (jnp.where(lane == 0, num, 0.0)
                    + jnp.where(lane == 1, den, 0.0)
                    + jnp.where(lane == 2, npred, 0.0))


@jax.jit
def kernel(logits, target):
    lg = jnp.asarray(logits, dtype=jnp.float32)
    tg = jnp.asarray(target).astype(jnp.int32)
    bsz = lg.shape[0]

    dist, dgt = pl.pallas_call(
        _dist_kernel,
        out_shape=[
            jax.ShapeDtypeStruct((bsz, _H, _W), jnp.float32),
            jax.ShapeDtypeStruct((bsz, _H, _W), jnp.int32),
        ],
        scratch_shapes=[pltpu.VMEM((_W, _B * _H), jnp.float32)],
    )(tg)

    klc, contrib = pl.pallas_call(
        _kl_kernel,
        grid=(bsz,),
        in_specs=[
            pl.BlockSpec((1, _C, _H, _W), lambda b: (b, 0, 0, 0)),
            pl.BlockSpec((1, _H, _W), lambda b: (b, 0, 0)),
            pl.BlockSpec((1, _H, _W), lambda b: (b, 0, 0)),
        ],
        out_specs=[
            pl.BlockSpec((1, _H, _W), lambda b: (b, 0, 0)),
            pl.BlockSpec((1, _H, _W), lambda b: (b, 0, 0)),
        ],
        out_shape=[
            jax.ShapeDtypeStruct((bsz, _H, _W), jnp.float32),
            jax.ShapeDtypeStruct((bsz, _H, _W), jnp.float32),
        ],
    )(lg, dist, dgt)

    eps_table = jnp.asarray(_EPS_TABLE, dtype=jnp.float32).reshape(1, _NEPS)

    sums = pl.pallas_call(
        _loss_kernel,
        in_specs=[
            pl.BlockSpec(memory_space=pltpu.SMEM),
            pl.BlockSpec(memory_space=pltpu.VMEM),
            pl.BlockSpec(memory_space=pltpu.VMEM),
            pl.BlockSpec(memory_space=pltpu.VMEM),
        ],
        out_shape=jax.ShapeDtypeStruct((1, 128), jnp.float32),
    )(eps_table, klc, contrib, dgt)

    num = sums[0, 0]
    den = sums[0, 1]
    npred = sums[0, 2]
    return jnp.where(npred >= 1.0, num / den, jnp.zeros(()))


# speedup vs baseline: 1.0399x; 1.0399x over previous
import functools

import jax
import jax.numpy as jnp
import numpy as np
from jax import lax
from jax.experimental import pallas as pl
from jax.experimental.pallas import tpu as pltpu

_IGNORE_LABEL = 255
_MAX_N_RATIO = 1.0 / 100
_LB_SMOOTH = 0.2
_MAX_CLIP_DIST = 20.0

_B = 4
_H = 224
_W = 224
_C = 19
_NDIR = 7

_NEPS = 160


def _build_eps_table():
    out = []
    e = np.float32(1e-5)
    for _ in range(_NEPS):
        out.append(float(e))
        e = np.float32(e * np.float32(1.2))
    return out


_EPS_TABLE = _build_eps_table()

_BIG = 1.0e9

_DX = (1, -1, 0, 0, -1, 1, -1, 1, 0)
_DY = (0, 0, -1, 1, 1, 1, -1, -1, 0)

_KUNROLL = 8


def _pad2d(x, value):
    h, w = x.shape
    row = jnp.full((1, w), value, dtype=x.dtype)
    x = jnp.concatenate([row, x, row], axis=0)
    col = jnp.full((h + 2, 1), value, dtype=x.dtype)
    return jnp.concatenate([col, x, col], axis=1)


def _pad3d(x, value):
    b, h, w = x.shape
    row = jnp.full((b, 1, w), value, dtype=x.dtype)
    x = jnp.concatenate([row, x, row], axis=1)
    col = jnp.full((b, h + 2, 1), value, dtype=x.dtype)
    return jnp.concatenate([col, x, col], axis=2)


def _dist_kernel(tg_ref, dist_ref, dgt_ref, g2t_ref):
    gt = tg_ref[...].astype(jnp.int32)

    dud = gt[:, 1:, :] - gt[:, :-1, :]
    dlr = gt[:, :, 1:] - gt[:, :, :-1]
    zrow = jnp.zeros((_B, 1, _W), dtype=jnp.int32)
    zcol = jnp.zeros((_B, _H, 1), dtype=jnp.int32)
    b_ud = jnp.concatenate([dud, zrow], axis=1) != 0
    b_lr = jnp.concatenate([dlr, zcol], axis=2) != 0
    bnd = b_ud | b_lr | (gt == _IGNORE_LABEL)
    pos = jnp.logical_not(bnd)

    init = jnp.where(pos, 1.0e6, 0.0).astype(jnp.float32)
    fwd = init
    bwd = init
    t = 1
    while t < _H:
        pad = jnp.full((_B, t, _W), _BIG, dtype=jnp.float32)
        fwd = jnp.minimum(
            fwd, jnp.concatenate([pad, fwd[:, :-t, :]], axis=1) + t)
        bwd = jnp.minimum(
            bwd, jnp.concatenate([bwd[:, t:, :], pad], axis=1) + t)
        t *= 2
    d0 = jnp.minimum(fwd, bwd)
    g2 = d0 * d0

    g2t_ref[...] = jnp.concatenate(
        [jnp.transpose(g2[b]) for b in range(_B)], axis=1)
    jf = lax.broadcasted_iota(jnp.int32, (_W, 1), 0).astype(jnp.float32)

    def body(tk, acc):
        k0 = pl.multiple_of(tk * _KUNROLL, _KUNROLL)
        rows = g2t_ref[pl.ds(k0, _KUNROLL), :]
        k0f = k0.astype(jnp.float32)
        for u in range(_KUNROLL):
            off = (jf - (k0f + u)) ** 2
            acc = jnp.minimum(acc, off + rows[u:u + 1, :])
        return acc

    acc0 = jnp.full((_W, _B * _H), 1.0e30, dtype=jnp.float32)
    d2t = lax.fori_loop(0, _W // _KUNROLL, body, acc0)
    dist = jnp.stack(
        [jnp.transpose(jnp.sqrt(d2t[:, b * _H:(b + 1) * _H]))
         for b in range(_B)], axis=0)

    dmap = jnp.where(pos, jnp.maximum(dist - 1.0, 0.0), 0.0)

    pdm = _pad3d(dmap, 1.0e5)
    best = pdm[:, 1 + _DX[0]:1 + _DX[0] + _H, 1 + _DY[0]:1 + _DY[0] + _W]
    idx = jnp.zeros((_B, _H, _W), dtype=jnp.int32)
    for d in range(1, 9):
        v = pdm[:, 1 + _DX[d]:1 + _DX[d] + _H, 1 + _DY[d]:1 + _DY[d] + _W]
        better = v < best
        idx = jnp.where(better, jnp.int32(d), idx)
        best = jnp.minimum(best, v)

    dist_ref[...] = dmap
    dgt_ref[...] = idx


def _kl_kernel(lg_ref, klc_ref, dp_ref):
    x = lg_ref[0]

    m = jnp.max(x, axis=0, keepdims=True)
    sh = x - m
    e = jnp.exp(sh)
    se = jnp.sum(e, axis=0, keepdims=True)
    ls = sh - jnp.log(se)
    pk = jnp.exp(ls)

    kud = jnp.sum(pk[:, :-1, :] * (ls[:, :-1, :] - ls[:, 1:, :]), axis=0)
    klr = jnp.sum(pk[:, :, :-1] * (ls[:, :, :-1] - ls[:, :, 1:]), axis=0)
    zrow = jnp.zeros((1, _W), dtype=jnp.float32)
    zcol = jnp.zeros((_H, 1), dtype=jnp.float32)
    klc_ref[0] = (jnp.concatenate([kud, zrow], axis=0)
                  + jnp.concatenate([klr, zcol], axis=1))

    unif = 1.0 / _C
    lunif = -float(np.log(np.float32(_C)))
    s = jnp.sum(pk * ls, axis=0)
    spad = _pad2d(s, lunif)
    h2 = _H + 2
    prow = jnp.full((_C, 1, _W), unif, dtype=jnp.float32)
    ppad = jnp.concatenate([prow, pk, prow], axis=1)
    pcol = jnp.full((_C, h2, 1), unif, dtype=jnp.float32)
    ppad = jnp.concatenate([pcol, ppad, pcol], axis=2)

    dps = []
    for d in range(_NDIR):
        dx, dy = _DX[d], _DY[d]
        sn = spad[1 + dx:1 + dx + _H, 1 + dy:1 + dy + _W]
        pn = ppad[:, 1 + dx:1 + dx + _H, 1 + dy:1 + dy + _W]
        dps.append(sn - jnp.sum(pn * ls, axis=0))
    dp_ref[0] = jnp.stack(dps, axis=0)


def _loss_kernel(tbl_ref, klc_ref, dp_ref, dist_ref, dgt_ref, out_ref):
    klc = klc_ref[...]

    max_n = _H * _W * _MAX_N_RATIO

    def bs_body(_, lohi):
        lo, hi = lohi
        mid = (lo + hi) // 2
        cnt = jnp.sum((klc > tbl_ref[0, mid]).astype(jnp.float32))
        big = cnt > max_n
        return (jnp.where(big, mid + 1, lo), jnp.where(big, hi, mid))

    lo, _hi = lax.fori_loop(
        0, 8, bs_body, (jnp.int32(0), jnp.int32(_NEPS)))
    eps = tbl_ref[0, jnp.minimum(lo, _NEPS - 1)]

    bp = _pad3d((klc > eps).astype(jnp.float32), 0.0)
    edge = jnp.zeros((_B, _H, _W), dtype=jnp.float32)
    for dx in (-1, 0, 1):
        for dy in (-1, 0, 1):
            edge = edge + bp[:, 1 + dx:1 + dx + _H, 1 + dy:1 + dy + _W]
    pred = edge > 0.0

    dgt = dgt_ref[...]
    sel = jnp.logical_and(pred, dgt != 8).astype(jnp.float32)

    lb_pos = 1.0 - _LB_SMOOTH
    lb_neg = _LB_SMOOTH / _NDIR
    num = jnp.float32(0.0)
    den = jnp.float32(0.0)
    npred = jnp.float32(0.0)
    for b in range(_B):
        x = dp_ref[b]
        m7 = jnp.max(x, axis=0, keepdims=True)
        shb = x - m7
        lse = jnp.log(jnp.sum(jnp.exp(shb), axis=0, keepdims=True))
        logs = shb - lse
        sumlogs = jnp.sum(logs, axis=0)
        pick = jnp.zeros((_H, _W), dtype=jnp.float32)
        dgtb = dgt[b]
        for c in range(_NDIR):
            pick = pick + jnp.where(dgtb == c, logs[c], 0.0)
        loss_vec = -(lb_neg * sumlogs + (lb_pos - lb_neg) * pick)
        w = jnp.clip(dist_ref[b], 0.0, _MAX_CLIP_DIST) / _MAX_CLIP_DIST
        selb = sel[b]
        num = num + jnp.sum(loss_vec * w * selb)
        den = den + jnp.sum(selb)
        npred = npred + jnp.sum(pred[b].astype(jnp.float32))

    lane = lax.broadcasted_iota(jnp.int32, (1, 128), 1)
    out_ref[...] = (jnp.where(lane == 0, num, 0.0)
                    + jnp.where(lane == 1, den, 0.0)
                    + jnp.where(lane == 2, npred, 0.0))


@jax.jit
def kernel(logits, target):
    lg = jnp.asarray(logits, dtype=jnp.float32)
    tg = jnp.asarray(target).astype(jnp.int32)
    bsz = lg.shape[0]

    dist, dgt = pl.pallas_call(
        _dist_kernel,
        out_shape=[
            jax.ShapeDtypeStruct((bsz, _H, _W), jnp.float32),
            jax.ShapeDtypeStruct((bsz, _H, _W), jnp.int32),
        ],
        scratch_shapes=[pltpu.VMEM((_W, _B * _H), jnp.float32)],
    )(tg)

    klc, dp = pl.pallas_call(
        _kl_kernel,
        grid=(bsz,),
        in_specs=[pl.BlockSpec((1, _C, _H, _W), lambda b: (b, 0, 0, 0))],
        out_specs=[
            pl.BlockSpec((1, _H, _W), lambda b: (b, 0, 0)),
            pl.BlockSpec((1, _NDIR, _H, _W), lambda b: (b, 0, 0, 0)),
        ],
        out_shape=[
            jax.ShapeDtypeStruct((bsz, _H, _W), jnp.float32),
            jax.ShapeDtypeStruct((bsz, _NDIR, _H, _W), jnp.float32),
        ],
    )(lg)

    eps_table = jnp.asarray(_EPS_TABLE, dtype=jnp.float32).reshape(1, _NEPS)

    sums = pl.pallas_call(
        _loss_kernel,
        in_specs=[
            pl.BlockSpec(memory_space=pltpu.SMEM),
            pl.BlockSpec(memory_space=pltpu.VMEM),
            pl.BlockSpec(memory_space=pltpu.VMEM),
            pl.BlockSpec(memory_space=pltpu.VMEM),
            pl.BlockSpec(memory_space=pltpu.VMEM),
        ],
        out_shape=jax.ShapeDtypeStruct((1, 128), jnp.float32),
    )(eps_table, klc, dp, dist, dgt)

    num = sums[0, 0]
    den = sums[0, 1]
    npred = sums[0, 2]
    return jnp.where(npred >= 1.0, num / den, jnp.zeros(()))
